# baseline (speedup 1.0000x reference)
import functools

import jax
import jax.numpy as jnp
from jax import lax
from jax.experimental import pallas as pl
from jax.experimental.pallas import tpu as pltpu
from jax.experimental.pallas import tpu_sc as plsc

HIDDEN = 64
BATCH = 16384
CHUNK = 128


def kernel(labels, embedding_table):
    info = plsc.get_sparse_core_info()
    nc, ns = info.num_cores, info.num_subcores
    nw = nc * ns
    b_per_w = BATCH // nw
    n_chunks = b_per_w // CHUNK

    mesh = plsc.VectorSubcoreMesh(core_axis_name="c", subcore_axis_name="s")

    @functools.partial(
        pl.kernel,
        mesh=mesh,
        out_type=jax.ShapeDtypeStruct((BATCH, HIDDEN), jnp.float32),
        scratch_types=[
            pltpu.VMEM((n_chunks, CHUNK), jnp.int32),
            pltpu.VMEM((b_per_w, HIDDEN), jnp.float32),
            pltpu.SemaphoreType.DMA,
        ],
        compiler_params=pltpu.CompilerParams(use_tc_tiling_on_sc=False),
    )
    def emb(labels_hbm, table_hbm, out_hbm, idx_v, rows_v, sem):
        wid = lax.axis_index("s") * nc + lax.axis_index("c")
        pltpu.sync_copy(labels_hbm.at[wid], idx_v)
        copies = []
        for j in range(n_chunks):
            copies.append(
                pltpu.async_copy(
                    table_hbm.at[idx_v.at[j]],
                    rows_v.at[pl.ds(j * CHUNK, CHUNK)],
                    sem,
                )
            )
        for c in copies:
            c.wait()
        pltpu.sync_copy(rows_v, out_hbm.at[pl.ds(wid * b_per_w, b_per_w)])

    labels_grouped = labels.reshape(nw, n_chunks, CHUNK)
    return emb(labels_grouped, embedding_table)


# speedup vs baseline: 1.5436x; 1.5436x over previous
import functools

import jax
import jax.numpy as jnp
from jax import lax
from jax.experimental import pallas as pl
from jax.experimental.pallas import tpu as pltpu
from jax.experimental.pallas import tpu_sc as plsc

HIDDEN = 64
BATCH = 16384
K = 16


def kernel(labels, embedding_table):
    info = plsc.get_sparse_core_info()
    nc, ns = info.num_cores, info.num_subcores
    nw = nc * ns
    b_per_w = BATCH // nw
    n_groups = b_per_w // K

    mesh = plsc.VectorSubcoreMesh(core_axis_name="c", subcore_axis_name="s")

    @functools.partial(
        pl.kernel,
        mesh=mesh,
        out_type=jax.ShapeDtypeStruct((BATCH, 128), jnp.float32),
        scratch_types=[
            pltpu.VMEM_SHARED((ns, b_per_w), jnp.int32),
            pltpu.SMEM((b_per_w,), jnp.int32),
            pltpu.VMEM((K, 8, HIDDEN), jnp.float32),
            pltpu.VMEM((b_per_w, 128), jnp.float32),
            pltpu.SemaphoreType.DMA,
        ],
        compiler_params=pltpu.CompilerParams(use_tc_tiling_on_sc=True),
    )
    def emb(labels_hbm, table_hbm, out_hbm, idx_sh, idx_s, stage_v, rows_v, sem):
        sid = lax.axis_index("s")
        wid = sid * nc + lax.axis_index("c")
        base = wid * b_per_w
        pltpu.sync_copy(labels_hbm.at[pl.ds(base, b_per_w)], idx_sh.at[sid])
        pltpu.sync_copy(idx_sh.at[sid], idx_s)

        def group(g, carry):
            copies = []
            for j in range(K):
                lbl = idx_s[g * K + j]
                t = (lbl // 8) * 8
                copies.append(
                    pltpu.async_copy(
                        table_hbm.at[pl.ds(t, 8), :], stage_v.at[j], sem
                    )
                )
            for j in range(K):
                copies[j].wait()
                lbl = idx_s[g * K + j]
                c = lbl % 8
                r = g * K + j
                for k in range(HIDDEN // 16):
                    rows_v[r, pl.ds(16 * k, 16)] = stage_v[j, c, pl.ds(16 * k, 16)]
            return carry

        lax.fori_loop(0, n_groups, group, 0)
        pltpu.sync_copy(rows_v, out_hbm.at[pl.ds(base, b_per_w), :])

    out = emb(labels, embedding_table)
    return out[:, :HIDDEN]
